# baseline (speedup 1.0000x reference)
import jax
import jax.numpy as jnp
from jax.experimental import pallas as pl
from jax.experimental.pallas import tpu as pltpu

B, N, CIN, COUT = 2, 2048, 128, 128
BS = 128
PAIR = 2 * BS
NS = N // PAIR
BQ = 512
NQ = N // BQ
Q0 = NS + 1
PROP_DTYPE = jnp.float8_e4m3fn


def _fused_body(adj0_ref, adj1_ref, x_ref, w_ref, b_ref, out_ref,
                adjb_s, d_s, ps_s, u_s):
    s = pl.program_id(0)

    @pl.when(s < NS)
    def _():
        ones = jnp.ones((N, 128), PROP_DTYPE)
        for h, aref in enumerate((adj0_ref, adj1_ref)):
            rows = pl.ds(s * PAIR + h * BS, BS)
            af8 = aref[...].astype(PROP_DTYPE)
            adjb_s[rows, :] = af8
            deg = jnp.dot(af8, ones, preferred_element_type=jnp.float32)
            d = jax.lax.rsqrt(deg[:, :1])
            d_s[rows, :] = d
            for b in range(B):
                u_s[rows, b * CIN:(b + 1) * CIN] = (
                    x_ref[b, rows, :] * d).astype(PROP_DTYPE)

    for t in range(NS):
        @pl.when(s == t + 1)
        def _(t=t):
            hi = (t + 1) * PAIR
            ps_s[t * PAIR:hi, :] = jnp.dot(
                adjb_s[t * PAIR:hi, :hi], u_s[:hi, :],
                preferred_element_type=jnp.float32)
            if t > 0:
                ps_s[:t * PAIR, :] += jnp.dot(
                    adjb_s[:t * PAIR, t * PAIR:hi], u_s[t * PAIR:hi, :],
                    preferred_element_type=jnp.float32)

    @pl.when(s == NS)
    def _():
        dd = d_s[...]
        for b in range(B):
            sl = slice(b * CIN, (b + 1) * CIN)
            u_s[:, sl] = ((x_ref[b] - ps_s[:, sl] * dd) * dd).astype(
                PROP_DTYPE)

    @pl.when(s >= Q0)
    def _():
        rows = pl.ds((s - Q0) * BQ, BQ)
        d = d_s[rows, :]
        acc = jnp.dot(adjb_s[rows, :], u_s[...],
                      preferred_element_type=jnp.float32)
        q = (acc * d).astype(jnp.bfloat16)
        p = (ps_s[rows, :] * d).astype(jnp.bfloat16)
        w0 = w_ref[0, 0]
        w1 = w_ref[1, 0]
        w2 = w_ref[2, 0]
        wa = (w0 + w1 + w2).astype(jnp.bfloat16)
        wb = (-(w1 + 2.0 * w2)).astype(jnp.bfloat16)
        wc = (-2.0 * w2).astype(jnp.bfloat16)
        bias = b_ref[0, 0, :]
        for b in range(B):
            sl = slice(b * CIN, (b + 1) * CIN)
            xb = x_ref[b, rows, :].astype(jnp.bfloat16)
            r = jnp.dot(xb, wa, preferred_element_type=jnp.float32)
            r = r + jnp.dot(p[:, sl], wb, preferred_element_type=jnp.float32)
            r = r + jnp.dot(q[:, sl], wc, preferred_element_type=jnp.float32)
            out_ref[b] = jnp.maximum(r + bias, 0.0)


@jax.jit
def kernel(x, adj, weight, bias):
    return pl.pallas_call(
        _fused_body,
        grid=(Q0 + NQ,),
        in_specs=[
            pl.BlockSpec((BS, N), lambda s: (jnp.minimum(2 * s, 2 * NS - 2), 0)),
            pl.BlockSpec((BS, N), lambda s: (jnp.minimum(2 * s + 1, 2 * NS - 1), 0)),
            pl.BlockSpec((B, N, CIN), lambda s: (0, 0, 0)),
            pl.BlockSpec((3, 1, CIN, COUT), lambda s: (0, 0, 0, 0)),
            pl.BlockSpec((1, 1, COUT), lambda s: (0, 0, 0)),
        ],
        out_specs=pl.BlockSpec(
            (B, BQ, COUT), lambda s: (0, jnp.maximum(s - Q0, 0), 0)),
        out_shape=jax.ShapeDtypeStruct((B, N, COUT), jnp.float32),
        scratch_shapes=[
            pltpu.VMEM((N, N), PROP_DTYPE),
            pltpu.VMEM((N, 1), jnp.float32),
            pltpu.VMEM((N, B * CIN), jnp.float32),
            pltpu.VMEM((N, B * CIN), PROP_DTYPE),
        ],
    )(adj, adj, x, weight, bias)


# speedup vs baseline: 1.2101x; 1.2101x over previous
import jax
import jax.numpy as jnp
from jax.experimental import pallas as pl
from jax.experimental.pallas import tpu as pltpu

B, N, CIN, COUT = 2, 2048, 128, 128
BS = 256
PAIR = 2 * BS
NS = N // PAIR
BQ = 1024
NQ = N // BQ
Q0 = NS + 1
PROP_DTYPE = jnp.float8_e4m3fn


def _fused_body(adj0_ref, adj1_ref, x_ref, w_ref, b_ref, out_ref,
                adjb_s, d_s, ps_s, u_s):
    s = pl.program_id(0)

    @pl.when(s < NS)
    def _():
        ones = jnp.ones((N, 128), PROP_DTYPE)
        for h, aref in enumerate((adj0_ref, adj1_ref)):
            rows = pl.ds(s * PAIR + h * BS, BS)
            af8 = aref[...].astype(PROP_DTYPE)
            adjb_s[rows, :] = af8
            deg = jnp.dot(af8, ones, preferred_element_type=jnp.float32)
            d = jax.lax.rsqrt(deg[:, :1])
            d_s[rows, :] = d
            for b in range(B):
                u_s[rows, b * CIN:(b + 1) * CIN] = (
                    x_ref[b, rows, :] * d).astype(PROP_DTYPE)

    for t in range(NS):
        @pl.when(s == t + 1)
        def _(t=t):
            hi = (t + 1) * PAIR
            ps_s[t * PAIR:hi, :] = jnp.dot(
                adjb_s[t * PAIR:hi, :hi], u_s[:hi, :],
                preferred_element_type=jnp.float32)
            if t > 0:
                ps_s[:t * PAIR, :] += jnp.dot(
                    adjb_s[:t * PAIR, t * PAIR:hi], u_s[t * PAIR:hi, :],
                    preferred_element_type=jnp.float32)

    @pl.when(s == NS)
    def _():
        dd = d_s[...]
        for b in range(B):
            sl = slice(b * CIN, (b + 1) * CIN)
            u_s[:, sl] = ((x_ref[b] - ps_s[:, sl] * dd) * dd).astype(
                PROP_DTYPE)

    @pl.when(s >= Q0)
    def _():
        rows = pl.ds((s - Q0) * BQ, BQ)
        d = d_s[rows, :]
        acc = jnp.dot(adjb_s[rows, :], u_s[...],
                      preferred_element_type=jnp.float32)
        q = (acc * d).astype(jnp.bfloat16)
        p = (ps_s[rows, :] * d).astype(jnp.bfloat16)
        w0 = w_ref[0, 0]
        w1 = w_ref[1, 0]
        w2 = w_ref[2, 0]
        wa = (w0 + w1 + w2).astype(jnp.bfloat16)
        wb = (-(w1 + 2.0 * w2)).astype(jnp.bfloat16)
        wc = (-2.0 * w2).astype(jnp.bfloat16)
        bias = b_ref[0, 0, :]
        for b in range(B):
            sl = slice(b * CIN, (b + 1) * CIN)
            xb = x_ref[b, rows, :].astype(jnp.bfloat16)
            r = jnp.dot(xb, wa, preferred_element_type=jnp.float32)
            r = r + jnp.dot(p[:, sl], wb, preferred_element_type=jnp.float32)
            r = r + jnp.dot(q[:, sl], wc, preferred_element_type=jnp.float32)
            out_ref[b] = jnp.maximum(r + bias, 0.0)


@jax.jit
def kernel(x, adj, weight, bias):
    return pl.pallas_call(
        _fused_body,
        grid=(Q0 + NQ,),
        in_specs=[
            pl.BlockSpec((BS, N), lambda s: (jnp.minimum(2 * s, 2 * NS - 2), 0)),
            pl.BlockSpec((BS, N), lambda s: (jnp.minimum(2 * s + 1, 2 * NS - 1), 0)),
            pl.BlockSpec((B, N, CIN), lambda s: (0, 0, 0)),
            pl.BlockSpec((3, 1, CIN, COUT), lambda s: (0, 0, 0, 0)),
            pl.BlockSpec((1, 1, COUT), lambda s: (0, 0, 0)),
        ],
        out_specs=pl.BlockSpec(
            (B, BQ, COUT), lambda s: (0, jnp.maximum(s - Q0, 0), 0)),
        out_shape=jax.ShapeDtypeStruct((B, N, COUT), jnp.float32),
        scratch_shapes=[
            pltpu.VMEM((N, N), PROP_DTYPE),
            pltpu.VMEM((N, 1), jnp.float32),
            pltpu.VMEM((N, B * CIN), jnp.float32),
            pltpu.VMEM((N, B * CIN), PROP_DTYPE),
        ],
    )(adj, adj, x, weight, bias)
